# baseline (speedup 1.0000x reference)
import math

import jax
import jax.numpy as jnp
from jax import lax
from jax.experimental import pallas as pl
from jax.experimental.pallas import tpu as pltpu
from jax.experimental.pallas import tpu_sc as plsc

_NC = 2
_NS = 16
_NW = _NC * _NS
_LANES = 16


def _sc_partial_sums(x128, cols_idx):
    m = x128.shape[0]
    rows = m // _NW

    mesh = plsc.VectorSubcoreMesh(core_axis_name="c", subcore_axis_name="s")

    def body(x_hbm, cols_hbm, out_hbm, cols_v, chunk_v, stage_v):
        c = lax.axis_index("c")
        s = lax.axis_index("s")
        wid = s * _NC + c
        pltpu.sync_copy(cols_hbm, cols_v)
        pltpu.sync_copy(x_hbm.at[pl.ds(wid * rows, rows), :], chunk_v)

        cols = cols_v[...]
        row0 = jnp.zeros((_LANES,), jnp.int32)
        acc0 = jnp.zeros((_LANES,), jnp.float32)

        def step(_, carry):
            row, acc = carry
            g = plsc.load_gather(chunk_v, [row, cols])
            return (row + 1, acc + g)

        _, acc = lax.fori_loop(0, rows, step, (row0, acc0))
        stage_v[...] = acc
        pltpu.sync_copy(stage_v, out_hbm.at[wid])

    run = pl.kernel(
        body,
        mesh=mesh,
        out_type=jax.ShapeDtypeStruct((_NW, _LANES), jnp.float32),
        scratch_types=[
            pltpu.VMEM((_LANES,), jnp.int32),
            pltpu.VMEM((rows, 128), jnp.float32),
            pltpu.VMEM((_LANES,), jnp.float32),
        ],
        compiler_params=pltpu.CompilerParams(needs_layout_passes=False),
    )
    return run(x128, cols_idx)


def _tc_finish(partials, log_uniform, inv_scale):
    def body(p_ref, o_ref):
        total = log_uniform + inv_scale * jnp.sum(p_ref[...])
        o_ref[...] = jnp.full((1, 1), 0.0, jnp.float32) + total

    return pl.pallas_call(
        body,
        out_shape=jax.ShapeDtypeStruct((1, 1), jnp.float32),
    )(partials)


def kernel(log_probs, top_k_indices, min_k_expert_indices, layer_idx):
    b, t, e = log_probs.shape
    n = b * t
    k = min_k_expert_indices.shape[0]

    x128 = log_probs.reshape(n * e // 128, 128)
    mink = min_k_expert_indices.astype(jnp.int32)
    cols = jnp.concatenate([mink, mink + e])

    partials = _sc_partial_sums(x128, cols)
    out = _tc_finish(partials, math.log(1.0 / k), -1.0 / (k * n))
    return out[0, 0]


# speedup vs baseline: 2.2000x; 2.2000x over previous
import math

import jax
import jax.numpy as jnp
from jax import lax
from jax.experimental import pallas as pl
from jax.experimental.pallas import tpu as pltpu
from jax.experimental.pallas import tpu_sc as plsc

_NC = 2
_NS = 16
_NW = _NC * _NS
_LANES = 16
_UNROLL = 8


def _sc_row_sums(xt, minkv, k):
    n_rows, t = xt.shape
    e = n_rows * k // _NW

    mesh = plsc.VectorSubcoreMesh(core_axis_name="c", subcore_axis_name="s")

    def body(x_hbm, minkv_hbm, out_hbm, minkv_v, row_v, stage_v):
        c = lax.axis_index("c")
        s = lax.axis_index("s")
        wid = s * _NC + c
        pltpu.sync_copy(minkv_hbm, minkv_v)

        lane = lax.iota(jnp.int32, _LANES)
        mv = minkv_v[...]
        j = wid % k
        mj = jnp.sum(jnp.where(lane == j, mv, 0))
        row = (wid // k) * e + mj
        pltpu.sync_copy(x_hbm.at[row], row_v)

        accs0 = tuple(jnp.zeros((_LANES,), jnp.float32) for _ in range(_UNROLL))

        def step(i, accs):
            base = i * (_LANES * _UNROLL)
            return tuple(
                a + row_v[pl.ds(base + u * _LANES, _LANES)]
                for u, a in enumerate(accs)
            )

        accs = lax.fori_loop(0, t // (_LANES * _UNROLL), step, accs0)
        acc = accs[0]
        for a in accs[1:]:
            acc = acc + a
        stage_v[...] = acc
        pltpu.sync_copy(stage_v, out_hbm.at[wid])

    run = pl.kernel(
        body,
        mesh=mesh,
        out_type=jax.ShapeDtypeStruct((_NW, _LANES), jnp.float32),
        scratch_types=[
            pltpu.VMEM((_LANES,), jnp.int32),
            pltpu.VMEM((t,), jnp.float32),
            pltpu.VMEM((_LANES,), jnp.float32),
        ],
        compiler_params=pltpu.CompilerParams(needs_layout_passes=False),
    )
    return run(xt, minkv)


def _tc_finish(partials, log_uniform, inv_scale):
    def body(p_ref, o_ref):
        total = log_uniform + inv_scale * jnp.sum(p_ref[...])
        o_ref[...] = jnp.full((1, 1), 0.0, jnp.float32) + total

    return pl.pallas_call(
        body,
        out_shape=jax.ShapeDtypeStruct((1, 1), jnp.float32),
    )(partials)


def kernel(log_probs, top_k_indices, min_k_expert_indices, layer_idx):
    b, t, e = log_probs.shape
    n = b * t
    k = min_k_expert_indices.shape[0]

    xt = log_probs.transpose(0, 2, 1).reshape(b * e, t)
    mink = min_k_expert_indices.astype(jnp.int32)
    minkv = jnp.concatenate([mink, mink])

    partials = _sc_row_sums(xt, minkv, k)
    out = _tc_finish(partials, math.log(1.0 / k), -1.0 / (k * n))
    return out[0, 0]


# speedup vs baseline: 2.3548x; 1.0704x over previous
import math

import jax
import jax.numpy as jnp
from jax import lax
from jax.experimental import pallas as pl
from jax.experimental.pallas import tpu as pltpu
from jax.experimental.pallas import tpu_sc as plsc

_NS = 16
_LANES = 16
_UNROLL = 8


def _sc_loss(xt, minkv, k, log_uniform, inv_scale):
    n_rows, t = xt.shape
    rows_needed = 2 * _NS
    e_per_b = n_rows * k // rows_needed

    mesh = plsc.VectorSubcoreMesh(
        core_axis_name="c", subcore_axis_name="s", num_cores=1
    )

    def body(x_hbm, minkv_hbm, out_hbm, minkv_v, row_a, row_b, stage_v, all_v, out_stage, shared):
        w = lax.axis_index("s")
        pltpu.sync_copy(minkv_hbm, minkv_v)

        lane = lax.iota(jnp.int32, _LANES)
        mv = minkv_v[...]

        def row_of(q):
            mj = jnp.sum(jnp.where(lane == q % k, mv, 0))
            return (q // k) * e_per_b + mj

        pltpu.sync_copy(x_hbm.at[row_of(w)], row_a)
        pltpu.sync_copy(x_hbm.at[row_of(w + _NS)], row_b)

        accs0 = tuple(jnp.zeros((_LANES,), jnp.float32) for _ in range(_UNROLL))

        def make_step(ref):
            def step(i, accs):
                base = i * (_LANES * _UNROLL)
                return tuple(
                    a + ref[pl.ds(base + u * _LANES, _LANES)]
                    for u, a in enumerate(accs)
                )
            return step

        n_it = t // (_LANES * _UNROLL)
        accs = lax.fori_loop(0, n_it, make_step(row_a), accs0)
        accs = lax.fori_loop(0, n_it, make_step(row_b), accs)
        acc = accs[0]
        for a in accs[1:]:
            acc = acc + a

        stage_v[...] = acc
        pltpu.sync_copy(stage_v, shared.at[w])
        plsc.subcore_barrier()

        @pl.when(w == 0)
        def _():
            pltpu.sync_copy(shared, all_v)
            tot = all_v[0]
            for i in range(1, _NS):
                tot = tot + all_v[i]
            total = jnp.sum(tot)
            res = log_uniform + inv_scale * total
            out_stage[...] = jnp.zeros((_LANES,), jnp.float32) + res
            pltpu.sync_copy(out_stage, out_hbm)

    run = pl.kernel(
        body,
        mesh=mesh,
        out_type=jax.ShapeDtypeStruct((_LANES,), jnp.float32),
        scratch_types=[
            pltpu.VMEM((_LANES,), jnp.int32),
            pltpu.VMEM((t,), jnp.float32),
            pltpu.VMEM((t,), jnp.float32),
            pltpu.VMEM((_LANES,), jnp.float32),
            pltpu.VMEM((_NS, _LANES), jnp.float32),
            pltpu.VMEM((_LANES,), jnp.float32),
            pltpu.VMEM_SHARED((_NS, _LANES), jnp.float32),
        ],
        compiler_params=pltpu.CompilerParams(needs_layout_passes=False),
    )
    return run(xt, minkv)


def kernel(log_probs, top_k_indices, min_k_expert_indices, layer_idx):
    b, t, e = log_probs.shape
    n = b * t
    k = min_k_expert_indices.shape[0]

    xt = log_probs.transpose(0, 2, 1).reshape(b * e, t)
    mink = min_k_expert_indices.astype(jnp.int32)
    minkv = jnp.concatenate([mink, mink])

    out = _sc_loss(xt, minkv, k, math.log(1.0 / k), -1.0 / (k * n))
    return out[0]
